# baseline (device time: 38165 ns/iter reference)
import os

import jax
import jax.numpy as jnp
from jax import lax
from jax.experimental import pallas as pl
from jax.experimental.pallas import tpu as pltpu

_VARIANT = os.environ.get("KERNEL_VARIANT", "full")

N_DEV = 8
M_PER = 512
K = 4096
N_TOT = 2048
N_PER = 256
NBLK = 8
AROW = 8


def kernel(x, w_mat):
    def body(x_ref, w_ref, out_ref, y_ref, q_ref, recv_ref, amax_ref,
             amax_smem, sc_send, sc_recv, ch_send, ch_recv):
        j = pl.program_id(0)
        my_i = lax.axis_index("i")

        if _VARIANT != "nocomm":
            @pl.when(j == 0)
            def _barrier():
                barrier_sem = pltpu.get_barrier_semaphore()
                for d in range(1, N_DEV):
                    t = lax.rem(my_i + d, N_DEV)
                    pl.semaphore_signal(
                        barrier_sem, inc=1, device_id=(t,),
                        device_id_type=pl.DeviceIdType.MESH,
                    )
                pl.semaphore_wait(barrier_sem, N_DEV - 1)

        yj = jnp.dot(x_ref[...], w_ref[...], preferred_element_type=jnp.float32)
        yj = jnp.maximum(yj, 0.0)
        y_ref[:, pl.ds(j * N_PER, N_PER)] = yj

        mj = jnp.max(yj)

        @pl.when(j == 0)
        def _init_amax():
            amax_smem[0] = mj

        @pl.when(j > 0)
        def _acc_amax():
            amax_smem[0] = jnp.maximum(amax_smem[0], mj)

        @pl.when(j == NBLK - 1)
        def _comm():
            a_loc = amax_smem[0]

            if _VARIANT == "full":
                amax_ref[pl.ds(my_i * AROW, AROW), :] = jnp.full(
                    (AROW, 128), a_loc, jnp.float32
                )
                for d in range(1, N_DEV):
                    t = lax.rem(my_i + d, N_DEV)
                    pltpu.make_async_remote_copy(
                        src_ref=amax_ref.at[pl.ds(my_i * AROW, AROW), :],
                        dst_ref=amax_ref.at[pl.ds(my_i * AROW, AROW), :],
                        send_sem=sc_send.at[d],
                        recv_sem=sc_recv.at[d],
                        device_id=(t,),
                        device_id_type=pl.DeviceIdType.MESH,
                    ).start()
                for d in range(1, N_DEV):
                    s = lax.rem(my_i - d + N_DEV, N_DEV)
                    pltpu.make_async_remote_copy(
                        src_ref=amax_ref.at[pl.ds(s * AROW, AROW), :],
                        dst_ref=amax_ref.at[pl.ds(s * AROW, AROW), :],
                        send_sem=sc_send.at[d],
                        recv_sem=sc_recv.at[d],
                        device_id=(s,),
                        device_id_type=pl.DeviceIdType.MESH,
                    ).wait_recv()
                amax_g = jnp.max(amax_ref[...])
            else:
                amax_g = a_loc
            scale = amax_g / 127.0
            inv_scale = 127.0 / amax_g

            def quant_chunk(col):
                q = jnp.round(y_ref[:, pl.ds(col, N_PER)] * inv_scale)
                return jnp.clip(q, -127.0, 127.0).astype(jnp.int8)

            for d in range(1, N_DEV) if _VARIANT != "nocomm" else []:
                t = lax.rem(my_i + d, N_DEV)
                q_ref[:, pl.ds(t * N_PER, N_PER)] = quant_chunk(t * N_PER)
                pltpu.make_async_remote_copy(
                    src_ref=q_ref.at[:, pl.ds(t * N_PER, N_PER)],
                    dst_ref=recv_ref.at[pl.ds(my_i * M_PER, M_PER), :],
                    send_sem=ch_send.at[d],
                    recv_sem=ch_recv.at[d],
                    device_id=(t,),
                    device_id_type=pl.DeviceIdType.MESH,
                ).start()

            own = quant_chunk(my_i * N_PER)
            out_ref[pl.ds(my_i * M_PER, M_PER), :] = (
                own.astype(jnp.float32) * scale
            )

            for d in range(1, N_DEV) if _VARIANT != "nocomm" else []:
                s = lax.rem(my_i - d + N_DEV, N_DEV)
                pltpu.make_async_remote_copy(
                    src_ref=q_ref.at[:, pl.ds(s * N_PER, N_PER)],
                    dst_ref=recv_ref.at[pl.ds(s * M_PER, M_PER), :],
                    send_sem=ch_send.at[d],
                    recv_sem=ch_recv.at[d],
                    device_id=(s,),
                    device_id_type=pl.DeviceIdType.MESH,
                ).wait_recv()
                out_ref[pl.ds(s * M_PER, M_PER), :] = (
                    recv_ref[pl.ds(s * M_PER, M_PER), :].astype(jnp.float32)
                    * scale
                )

            for d in range(1, N_DEV) if _VARIANT != "nocomm" else []:
                t = lax.rem(my_i + d, N_DEV)
                pltpu.make_async_remote_copy(
                    src_ref=q_ref.at[:, pl.ds(t * N_PER, N_PER)],
                    dst_ref=recv_ref.at[pl.ds(my_i * M_PER, M_PER), :],
                    send_sem=ch_send.at[d],
                    recv_sem=ch_recv.at[d],
                    device_id=(t,),
                    device_id_type=pl.DeviceIdType.MESH,
                ).wait_send()
                if _VARIANT == "full":
                    pltpu.make_async_remote_copy(
                        src_ref=amax_ref.at[pl.ds(my_i * AROW, AROW), :],
                        dst_ref=amax_ref.at[pl.ds(my_i * AROW, AROW), :],
                        send_sem=sc_send.at[d],
                        recv_sem=sc_recv.at[d],
                        device_id=(t,),
                        device_id_type=pl.DeviceIdType.MESH,
                    ).wait_send()

    return pl.pallas_call(
        body,
        grid=(NBLK,),
        in_specs=[
            pl.BlockSpec((M_PER, K), lambda j: (0, 0)),
            pl.BlockSpec((K, N_PER), lambda j: (0, j)),
        ],
        out_specs=pl.BlockSpec((N_DEV * M_PER, N_PER), lambda j: (0, 0)),
        out_shape=jax.ShapeDtypeStruct((N_DEV * M_PER, N_PER), jnp.float32),
        scratch_shapes=[
            pltpu.VMEM((M_PER, N_TOT), jnp.float32),
            pltpu.VMEM((M_PER, N_TOT), jnp.int8),
            pltpu.VMEM((N_DEV * M_PER, N_PER), jnp.int8),
            pltpu.VMEM((N_DEV * AROW, 128), jnp.float32),
            pltpu.SMEM((1,), jnp.float32),
            pltpu.SemaphoreType.DMA((N_DEV,)),
            pltpu.SemaphoreType.DMA((N_DEV,)),
            pltpu.SemaphoreType.DMA((N_DEV,)),
            pltpu.SemaphoreType.DMA((N_DEV,)),
        ],
        compiler_params=pltpu.CompilerParams(
            dimension_semantics=("arbitrary",),
            collective_id=None if _VARIANT == "nocomm" else 0,
        ),
    )(x, w_mat)


# device time: 37642 ns/iter; 1.0139x vs baseline; 1.0139x over previous
import jax
import jax.numpy as jnp
from jax import lax
from jax.experimental import pallas as pl
from jax.experimental.pallas import tpu as pltpu

N_DEV = 8
M_PER = 512
K = 4096
N_TOT = 2048
N_PER = 256
Q16 = 32512.0


def kernel(x, w_mat):
    def body(x_ref, w_ref, out_ref, wbuf, q16_ref, recv_ref,
             gsrc_ref, gmax_ref, wsem, ch_send, ch_recv, g_send, g_recv):
        my_i = lax.axis_index("i")

        def w_dma(k, slot):
            t = lax.rem(my_i + 1 + k, N_DEV)
            return pltpu.make_async_copy(
                w_ref.at[:, pl.ds(t * N_PER, N_PER)],
                wbuf.at[slot],
                wsem.at[slot],
            )

        w_dma(0, 0).start()
        barrier_sem = pltpu.get_barrier_semaphore()
        for d in range(1, N_DEV):
            t = lax.rem(my_i + d, N_DEV)
            pl.semaphore_signal(
                barrier_sem, inc=1, device_id=(t,),
                device_id_type=pl.DeviceIdType.MESH,
            )
        pl.semaphore_wait(barrier_sem, N_DEV - 1)

        for k in range(N_DEV):
            t = lax.rem(my_i + 1 + k, N_DEV)
            if k + 1 < N_DEV:
                w_dma(k + 1, (k + 1) % 2).start()
            w_dma(k, k % 2).wait()

            yj = jnp.dot(x_ref[...], wbuf[k % 2],
                         preferred_element_type=jnp.float32)
            yj = jnp.maximum(yj, 0.0)
            a_c = jnp.maximum(jnp.max(yj), 1e-30)
            gsrc_ref[pl.ds(k, 1), :] = jnp.full((1, 128), a_c, jnp.float32)
            q16 = jnp.round(yj * (Q16 / a_c)).astype(jnp.int16)

            if k < N_DEV - 1:
                q16_ref[:, pl.ds(t * N_PER, N_PER)] = q16
                pltpu.make_async_remote_copy(
                    src_ref=q16_ref.at[:, pl.ds(t * N_PER, N_PER)],
                    dst_ref=recv_ref.at[pl.ds(my_i * M_PER, M_PER), :],
                    send_sem=ch_send.at[k],
                    recv_sem=ch_recv.at[k],
                    device_id=(t,),
                    device_id_type=pl.DeviceIdType.MESH,
                ).start()
            else:
                recv_ref[pl.ds(my_i * M_PER, M_PER), :] = q16

        gmax_ref[pl.ds(my_i * N_DEV, N_DEV), :] = gsrc_ref[...]
        for d in range(1, N_DEV):
            t = lax.rem(my_i + d, N_DEV)
            pltpu.make_async_remote_copy(
                src_ref=gsrc_ref,
                dst_ref=gmax_ref.at[pl.ds(my_i * N_DEV, N_DEV), :],
                send_sem=g_send.at[d - 1],
                recv_sem=g_recv.at[d - 1],
                device_id=(t,),
                device_id_type=pl.DeviceIdType.MESH,
            ).start()
        for d in range(1, N_DEV):
            s = lax.rem(my_i - d + N_DEV, N_DEV)
            pltpu.make_async_remote_copy(
                src_ref=gsrc_ref,
                dst_ref=gmax_ref.at[pl.ds(s * N_DEV, N_DEV), :],
                send_sem=g_send.at[d - 1],
                recv_sem=g_recv.at[d - 1],
                device_id=(s,),
                device_id_type=pl.DeviceIdType.MESH,
            ).wait_recv()
        amax_g = jnp.max(gmax_ref[...])
        s_g = amax_g / 127.0

        def decode(rows, a_chunk):
            dec = recv_ref[pl.ds(rows, M_PER), :].astype(jnp.float32) * (
                a_chunk / (Q16 * s_g)
            )
            q8 = jnp.clip(jnp.round(dec), 0.0, 127.0)
            out_ref[pl.ds(rows, M_PER), :] = q8 * s_g

        decode(
            my_i * M_PER,
            jnp.max(gmax_ref[pl.ds(my_i * N_DEV + N_DEV - 1, 1), :]),
        )

        for k in range(N_DEV - 1):
            s = lax.rem(my_i - 1 - k + N_DEV, N_DEV)
            pltpu.make_async_remote_copy(
                src_ref=q16_ref.at[:, pl.ds(s * N_PER, N_PER)],
                dst_ref=recv_ref.at[pl.ds(s * M_PER, M_PER), :],
                send_sem=ch_send.at[k],
                recv_sem=ch_recv.at[k],
                device_id=(s,),
                device_id_type=pl.DeviceIdType.MESH,
            ).wait_recv()
            decode(s * M_PER, jnp.max(gmax_ref[pl.ds(s * N_DEV + k, 1), :]))

        for k in range(N_DEV - 1):
            t = lax.rem(my_i + 1 + k, N_DEV)
            pltpu.make_async_remote_copy(
                src_ref=q16_ref.at[:, pl.ds(t * N_PER, N_PER)],
                dst_ref=recv_ref.at[pl.ds(my_i * M_PER, M_PER), :],
                send_sem=ch_send.at[k],
                recv_sem=ch_recv.at[k],
                device_id=(t,),
                device_id_type=pl.DeviceIdType.MESH,
            ).wait_send()
            pltpu.make_async_remote_copy(
                src_ref=gsrc_ref,
                dst_ref=gmax_ref.at[pl.ds(my_i * N_DEV, N_DEV), :],
                send_sem=g_send.at[k],
                recv_sem=g_recv.at[k],
                device_id=(t,),
                device_id_type=pl.DeviceIdType.MESH,
            ).wait_send()

    return pl.pallas_call(
        body,
        in_specs=[
            pl.BlockSpec(memory_space=pltpu.MemorySpace.VMEM),
            pl.BlockSpec(memory_space=pl.ANY),
        ],
        out_specs=pl.BlockSpec(memory_space=pltpu.MemorySpace.VMEM),
        out_shape=jax.ShapeDtypeStruct((N_DEV * M_PER, N_PER), jnp.float32),
        scratch_shapes=[
            pltpu.VMEM((2, K, N_PER), jnp.float32),
            pltpu.VMEM((M_PER, N_TOT), jnp.int16),
            pltpu.VMEM((N_DEV * M_PER, N_PER), jnp.int16),
            pltpu.VMEM((N_DEV, 128), jnp.float32),
            pltpu.VMEM((N_DEV * N_DEV, 128), jnp.float32),
            pltpu.SemaphoreType.DMA((2,)),
            pltpu.SemaphoreType.DMA((N_DEV - 1,)),
            pltpu.SemaphoreType.DMA((N_DEV - 1,)),
            pltpu.SemaphoreType.DMA((N_DEV - 1,)),
            pltpu.SemaphoreType.DMA((N_DEV - 1,)),
        ],
        compiler_params=pltpu.CompilerParams(collective_id=0),
    )(x, w_mat)


# device time: 30248 ns/iter; 1.2617x vs baseline; 1.2444x over previous
import os

import jax
import jax.numpy as jnp
from jax import lax
from jax.experimental import pallas as pl
from jax.experimental.pallas import tpu as pltpu

_VARIANT = os.environ.get("KERNEL_VARIANT", "full")
_COMM = _VARIANT != "nocomm"

N_DEV = 8
M_PER = 512
K = 4096
N_TOT = 2048
N_PER = 256
Q16 = 32512.0


def kernel(x, w_mat):
    def body(x_ref, w_ref, out_ref, wbuf, q16_ref, recv_ref,
             gsrc_ref, gmax_ref, wsem, ch_send, ch_recv, g_send, g_recv):
        my_i = lax.axis_index("i")

        def w_dma(k, slot):
            t = lax.rem(my_i + 1 + k, N_DEV)
            return pltpu.make_async_copy(
                w_ref.at[:, pl.ds(t * N_PER, N_PER)],
                wbuf.at[slot],
                wsem.at[slot],
            )

        w_dma(0, 0).start()
        if _COMM:
            barrier_sem = pltpu.get_barrier_semaphore()
            for d in range(1, N_DEV):
                t = lax.rem(my_i + d, N_DEV)
                pl.semaphore_signal(
                    barrier_sem, inc=1, device_id=(t,),
                    device_id_type=pl.DeviceIdType.MESH,
                )
            pl.semaphore_wait(barrier_sem, N_DEV - 1)

        for k in range(N_DEV):
            t = lax.rem(my_i + 1 + k, N_DEV)
            if k + 1 < N_DEV:
                w_dma(k + 1, (k + 1) % 2).start()
            w_dma(k, k % 2).wait()

            yj = jnp.dot(x_ref[...], wbuf[k % 2],
                         preferred_element_type=jnp.float32)
            yj = jnp.maximum(yj, 0.0)
            a_c = jnp.maximum(jnp.max(yj), 1e-30)
            gsrc_ref[pl.ds(k, 1), :] = jnp.full((1, 128), a_c, jnp.float32)
            q16 = jnp.round(yj * (Q16 / a_c)).astype(jnp.int16)

            if k < N_DEV - 1:
                q16_ref[:, pl.ds(t * N_PER, N_PER)] = q16
                if _COMM:
                    pltpu.make_async_remote_copy(
                        src_ref=q16_ref.at[:, pl.ds(t * N_PER, N_PER)],
                        dst_ref=recv_ref.at[pl.ds(my_i * M_PER, M_PER), :],
                        send_sem=ch_send.at[k],
                        recv_sem=ch_recv.at[k],
                        device_id=(t,),
                        device_id_type=pl.DeviceIdType.MESH,
                    ).start()
            else:
                recv_ref[pl.ds(my_i * M_PER, M_PER), :] = q16

        gmax_ref[pl.ds(my_i * N_DEV, N_DEV), :] = gsrc_ref[...]
        if _COMM:
            for d in range(1, N_DEV):
                t = lax.rem(my_i + d, N_DEV)
                pltpu.make_async_remote_copy(
                    src_ref=gsrc_ref,
                    dst_ref=gmax_ref.at[pl.ds(my_i * N_DEV, N_DEV), :],
                    send_sem=g_send.at[d - 1],
                    recv_sem=g_recv.at[d - 1],
                    device_id=(t,),
                    device_id_type=pl.DeviceIdType.MESH,
                ).start()
            for d in range(1, N_DEV):
                s = lax.rem(my_i - d + N_DEV, N_DEV)
                pltpu.make_async_remote_copy(
                    src_ref=gsrc_ref,
                    dst_ref=gmax_ref.at[pl.ds(s * N_DEV, N_DEV), :],
                    send_sem=g_send.at[d - 1],
                    recv_sem=g_recv.at[d - 1],
                    device_id=(s,),
                    device_id_type=pl.DeviceIdType.MESH,
                ).wait_recv()
            amax_g = jnp.max(gmax_ref[...])
        else:
            amax_g = jnp.max(gsrc_ref[...])
        s_g = amax_g / 127.0

        def decode(rows, a_chunk):
            dec = recv_ref[pl.ds(rows, M_PER), :].astype(jnp.float32) * (
                a_chunk / (Q16 * s_g)
            )
            q8 = jnp.clip(jnp.round(dec), 0.0, 127.0)
            out_ref[pl.ds(rows, M_PER), :] = q8 * s_g

        decode(
            my_i * M_PER,
            jnp.max(gmax_ref[pl.ds(my_i * N_DEV + N_DEV - 1, 1), :]),
        )

        for k in range(N_DEV - 1):
            s = lax.rem(my_i - 1 - k + N_DEV, N_DEV)
            if _COMM:
                pltpu.make_async_remote_copy(
                    src_ref=q16_ref.at[:, pl.ds(s * N_PER, N_PER)],
                    dst_ref=recv_ref.at[pl.ds(s * M_PER, M_PER), :],
                    send_sem=ch_send.at[k],
                    recv_sem=ch_recv.at[k],
                    device_id=(s,),
                    device_id_type=pl.DeviceIdType.MESH,
                ).wait_recv()
            decode(s * M_PER, jnp.max(gmax_ref[pl.ds(s * N_DEV + k, 1), :]))

        for k in range(N_DEV - 1) if _COMM else []:
            t = lax.rem(my_i + 1 + k, N_DEV)
            pltpu.make_async_remote_copy(
                src_ref=q16_ref.at[:, pl.ds(t * N_PER, N_PER)],
                dst_ref=recv_ref.at[pl.ds(my_i * M_PER, M_PER), :],
                send_sem=ch_send.at[k],
                recv_sem=ch_recv.at[k],
                device_id=(t,),
                device_id_type=pl.DeviceIdType.MESH,
            ).wait_send()
            pltpu.make_async_remote_copy(
                src_ref=gsrc_ref,
                dst_ref=gmax_ref.at[pl.ds(my_i * N_DEV, N_DEV), :],
                send_sem=g_send.at[k],
                recv_sem=g_recv.at[k],
                device_id=(t,),
                device_id_type=pl.DeviceIdType.MESH,
            ).wait_send()

    return pl.pallas_call(
        body,
        in_specs=[
            pl.BlockSpec(memory_space=pltpu.MemorySpace.VMEM),
            pl.BlockSpec(memory_space=pl.ANY),
        ],
        out_specs=pl.BlockSpec(memory_space=pltpu.MemorySpace.VMEM),
        out_shape=jax.ShapeDtypeStruct((N_DEV * M_PER, N_PER), jnp.float32),
        scratch_shapes=[
            pltpu.VMEM((2, K, N_PER), jnp.float32),
            pltpu.VMEM((M_PER, N_TOT), jnp.int16),
            pltpu.VMEM((N_DEV * M_PER, N_PER), jnp.int16),
            pltpu.VMEM((N_DEV, 128), jnp.float32),
            pltpu.VMEM((N_DEV * N_DEV, 128), jnp.float32),
            pltpu.SemaphoreType.DMA((2,)),
            pltpu.SemaphoreType.DMA((N_DEV - 1,)),
            pltpu.SemaphoreType.DMA((N_DEV - 1,)),
            pltpu.SemaphoreType.DMA((N_DEV - 1,)),
            pltpu.SemaphoreType.DMA((N_DEV - 1,)),
        ],
        compiler_params=pltpu.CompilerParams(
            collective_id=0 if _COMM else None,
        ),
    )(x, w_mat)
